# Initial kernel scaffold; baseline (speedup 1.0000x reference)
import jax
import jax.numpy as jnp
from jax import lax
from jax.experimental import pallas as pl
from jax.experimental.pallas import tpu as pltpu
from jax.experimental.pallas import tpu_sc as plsc

N = 10000
E = 160000
D = 256
K = 2
H = D // 2
NS = 16
NPAD = 10240
NTR = 8
NT = NPAD + NTR
EPT = E // NS
CH = 128
NCHUNK = 80
EPT_PAD = NCHUNK * CH
RPT = NPAD // NS
RC = 64
NRC = RPT // RC


def _rsqrt16(v):
    i = plsc.bitcast(v, jnp.int32)
    i = 0x5F3759DF - (i >> 1)
    y = plsc.bitcast(i, jnp.float32)
    for _ in range(3):
        y = y * (1.5 - 0.5 * v * y * y)
    return y


def _sc_kprop(x_flat, src_hbm, dst_hbm, out_hbm, h_hbm,
              sstage, dstage, dst2d, degout2d, gbuf, xb,
              dbuf, no_ref, ni_ref, cvec, zbuf, ones,
              agg, deg_out, deg_in, sem):
    cid = lax.axis_index("c")
    sid = lax.axis_index("s")
    lane = lax.broadcasted_iota(jnp.int32, (16,), 0)
    trash_vec = NPAD + (lane & (NTR - 1))
    r0 = sid * RPT
    g0 = sid * EPT
    cbase = cid * NPAD

    def _init_zb(i, _):
        zbuf[pl.ds(i * 16, 16)] = jnp.zeros((16,), jnp.float32)
        return _
    lax.fori_loop(0, RPT // 16, _init_zb, None)
    for t in range(8):
        ones[pl.ds(t * 16, 16)] = jnp.ones((16,), jnp.float32)
    pltpu.sync_copy(zbuf, deg_out.at[pl.ds(r0, RPT)])
    pltpu.sync_copy(zbuf, deg_in.at[pl.ds(r0, RPT)])

    @pl.when(sid == NS - 1)
    def _zero_trash():
        pltpu.sync_copy(zbuf.at[pl.ds(0, NTR)], deg_out.at[pl.ds(NPAD, NTR)])
        pltpu.sync_copy(zbuf.at[pl.ds(0, NTR)], deg_in.at[pl.ds(NPAD, NTR)])

    pltpu.sync_copy(src_hbm.at[pl.ds(g0, EPT)], sstage.at[pl.ds(0, EPT)])
    pltpu.sync_copy(dst_hbm.at[pl.ds(g0, EPT)], dstage.at[pl.ds(0, EPT)])

    def _edge_row(j, _):
        for t in range(8):
            o = j * CH + t * 16
            sv = sstage[pl.ds(o, 16)]
            dv = dstage[pl.ds(o, 16)]
            m = sv != dv
            dst2d[j, pl.ds(t * 16, 16)] = jnp.where(m, dv, trash_vec)
            degout2d[j, pl.ds(t * 16, 16)] = jnp.where(m, sv, trash_vec)
            sstage[pl.ds(o, 16)] = sv + cbase
        return _
    lax.fori_loop(0, EPT // CH - 1, _edge_row, None)
    o = (EPT // CH - 1) * CH
    sv = sstage[pl.ds(o, 16)]
    dv = dstage[pl.ds(o, 16)]
    m = sv != dv
    dst2d[EPT // CH - 1, pl.ds(0, 16)] = jnp.where(m, dv, trash_vec)
    degout2d[EPT // CH - 1, pl.ds(0, 16)] = jnp.where(m, sv, trash_vec)
    sstage[pl.ds(o, 16)] = sv + cbase
    for t in range(EPT // 16, EPT_PAD // 16):
        j, c = t // 8, (t % 8) * 16
        sstage[pl.ds(t * 16, 16)] = cbase + ((t - EPT // 16) * 16 + lane)
        dst2d[j, pl.ds(c, 16)] = trash_vec
        degout2d[j, pl.ds(c, 16)] = trash_vec

    plsc.subcore_barrier()

    def _deg_chunk(j, _):
        pltpu.sync_copy(ones, deg_out.at[degout2d.at[j]], add=True)
        pltpu.sync_copy(ones, deg_in.at[dst2d.at[j]], add=True)
        return _
    lax.fori_loop(0, NCHUNK, _deg_chunk, None)

    plsc.subcore_barrier()

    pltpu.sync_copy(deg_out.at[pl.ds(r0, RPT)], dbuf)

    def _no_chunk(i, _):
        v = dbuf[pl.ds(i * 16, 16)] + 1.0
        no_ref[pl.ds(i * 16, 16)] = _rsqrt16(v)
        return _
    lax.fori_loop(0, RPT // 16, _no_chunk, None)
    pltpu.sync_copy(deg_in.at[pl.ds(r0, RPT)], dbuf)

    def _ni_chunk(i, _):
        v = dbuf[pl.ds(i * 16, 16)] + 1.0
        ni = _rsqrt16(v)
        ni_ref[pl.ds(i * 16, 16)] = ni
        cvec[pl.ds(i * 16, 16)] = ni * no_ref[pl.ds(i * 16, 16)]
        return _
    lax.fori_loop(0, RPT // 16, _ni_chunk, None)

    def _dense(scale_ref, from_agg, to_slab, to_out):
        def _chunk(i, _):
            st = i * RC
            if from_agg:
                pltpu.sync_copy(agg.at[pl.ds(r0 + st, RC)], xb)
            else:
                pltpu.sync_copy(x_flat.at[pl.ds(cbase + r0 + st, RC)], xb)

            def _row(r, __):
                s = scale_ref[st + r]
                for c8 in range(H // 16):
                    xb[r, pl.ds(c8 * 16, 16)] = xb[r, pl.ds(c8 * 16, 16)] * s
                return __
            lax.fori_loop(0, RC, _row, None)
            if to_slab:
                pltpu.sync_copy(xb, h_hbm.at[pl.ds(cbase + r0 + st, RC)])
                pltpu.sync_copy(xb, agg.at[pl.ds(r0 + st, RC)])
            if to_out:
                pltpu.sync_copy(xb, out_hbm.at[pl.ds(cbase + r0 + st, RC)])
            return _
        lax.fori_loop(0, NRC, _chunk, None)

    _dense(no_ref, from_agg=False, to_slab=True, to_out=False)
    plsc.subcore_barrier()

    for k in range(K):
        def _edge_chunk(j, _):
            pltpu.async_copy(
                h_hbm.at[sstage.at[pl.ds(j * CH, CH)]], gbuf, sem).wait()
            pltpu.sync_copy(gbuf, agg.at[dst2d.at[j]], add=True)
            return _
        lax.fori_loop(0, NCHUNK, _edge_chunk, None)
        plsc.subcore_barrier()
        if k < K - 1:
            _dense(cvec, from_agg=True, to_slab=True, to_out=False)
        else:
            _dense(ni_ref, from_agg=True, to_slab=False, to_out=True)
        plsc.subcore_barrier()


@jax.jit
def kernel(x, edge_index):
    f32 = jnp.float32
    mesh = plsc.VectorSubcoreMesh(core_axis_name="c", subcore_axis_name="s")
    run = pl.kernel(
        _sc_kprop,
        out_type=(
            jax.ShapeDtypeStruct((2 * NPAD, H), f32),
            jax.ShapeDtypeStruct((2 * NPAD, H), f32),
        ),
        mesh=mesh,
        scratch_types=[
            pltpu.VMEM((EPT_PAD,), jnp.int32),
            pltpu.VMEM((EPT_PAD,), jnp.int32),
            pltpu.VMEM((NCHUNK, CH), jnp.int32),
            pltpu.VMEM((NCHUNK, CH), jnp.int32),
            pltpu.VMEM((CH, H), f32),
            pltpu.VMEM((RC, H), f32),
            pltpu.VMEM((RPT,), f32),
            pltpu.VMEM((RPT,), f32),
            pltpu.VMEM((RPT,), f32),
            pltpu.VMEM((RPT,), f32),
            pltpu.VMEM((RPT,), f32),
            pltpu.VMEM((CH,), f32),
            pltpu.VMEM_SHARED((NT, H), f32),
            pltpu.VMEM_SHARED((NT,), f32),
            pltpu.VMEM_SHARED((NT,), f32),
            pltpu.SemaphoreType.DMA,
        ],
    )
    xt = x.reshape(N, 2, H).transpose(1, 0, 2)
    xp = jnp.pad(xt, ((0, 0), (0, NPAD - N), (0, 0))).reshape(2 * NPAD, H)
    out_flat, _ = run(xp, edge_index[0], edge_index[1])
    out = out_flat.reshape(2, NPAD, H)[:, :N, :]
    return out.transpose(1, 0, 2).reshape(N, D)


# speedup vs baseline: 5.8997x; 5.8997x over previous
import jax
import jax.numpy as jnp
from jax import lax
from jax.experimental import pallas as pl
from jax.experimental.pallas import tpu as pltpu
from jax.experimental.pallas import tpu_sc as plsc

N = 10000
E = 160000
D = 256
K = 2
H = D // 2
NS = 16
NPAD = 10240
NTR = 8
NT = NPAD + NTR
EPT = E // NS
CH = 128
NCHUNK = 80
EPT_PAD = NCHUNK * CH
RPT = NPAD // NS
RC = 64
NRC = RPT // RC


def _rsqrt16(v):
    i = lax.bitcast_convert_type(v, jnp.int32)
    i = 0x5F3759DF - (i >> 1)
    y = lax.bitcast_convert_type(i, jnp.float32)
    for _ in range(3):
        y = y * (1.5 - 0.5 * v * y * y)
    return y


def _sc_kprop(x_flat, src_hbm, dst_hbm, out_hbm, h_hbm,
              sstage, dst2d, tmpd, degout_row, gbuf,
              dbuf, no_ref, ni_ref, cvec, ones,
              agg, deg_out, deg_in, sem):
    cid = lax.axis_index("c")
    sid = lax.axis_index("s")
    lane = lax.broadcasted_iota(jnp.int32, (16,), 0)
    trash_vec = NPAD + (lane & (NTR - 1))
    r0 = sid * RPT
    g0 = sid * EPT
    cbase = cid * NPAD

    def _init_zb(i, _):
        dbuf[pl.ds(i * 16, 16)] = jnp.zeros((16,), jnp.float32)
        return _
    lax.fori_loop(0, RPT // 16, _init_zb, None)
    for t in range(8):
        ones[pl.ds(t * 16, 16)] = jnp.ones((16,), jnp.float32)
    pltpu.sync_copy(dbuf, deg_out.at[pl.ds(r0, RPT)])
    pltpu.sync_copy(dbuf, deg_in.at[pl.ds(r0, RPT)])
    pltpu.sync_copy(src_hbm.at[pl.ds(g0, EPT)], sstage.at[pl.ds(0, EPT)])

    plsc.subcore_barrier()

    def _edge_row(j, _):
        pltpu.sync_copy(dst_hbm.at[pl.ds(g0 + j * CH, CH)], tmpd)
        for t in range(8):
            o = j * CH + t * 16
            sv = sstage[pl.ds(o, 16)]
            dv = tmpd[pl.ds(t * 16, 16)]
            m = sv != dv
            dst2d[j, pl.ds(t * 16, 16)] = jnp.where(m, dv, trash_vec)
            degout_row[0, pl.ds(t * 16, 16)] = jnp.where(m, sv, trash_vec)
            sstage[pl.ds(o, 16)] = sv + cbase
        pltpu.sync_copy(ones, deg_out.at[degout_row.at[0]], add=True)
        pltpu.sync_copy(ones, deg_in.at[dst2d.at[j]], add=True)
        return _
    lax.fori_loop(0, EPT // CH, _edge_row, None)
    jl = EPT // CH
    o = jl * CH
    sv = sstage[pl.ds(o, 16)]
    pltpu.sync_copy(dst_hbm.at[pl.ds(g0 + o, 16)], tmpd.at[pl.ds(0, 16)])
    dv = tmpd[pl.ds(0, 16)]
    m = sv != dv
    dst2d[jl, pl.ds(0, 16)] = jnp.where(m, dv, trash_vec)
    degout_row[0, pl.ds(0, 16)] = jnp.where(m, sv, trash_vec)
    sstage[pl.ds(o, 16)] = sv + cbase
    for t in range(EPT // 16, EPT_PAD // 16):
        j, c = t // 8, (t % 8) * 16
        sstage[pl.ds(t * 16, 16)] = cbase + ((t - EPT // 16) * 16 + lane)
        dst2d[j, pl.ds(c, 16)] = trash_vec
        if j == jl:
            degout_row[0, pl.ds(c, 16)] = trash_vec
    pltpu.sync_copy(ones, deg_out.at[degout_row.at[0]], add=True)
    pltpu.sync_copy(ones, deg_in.at[dst2d.at[jl]], add=True)

    plsc.subcore_barrier()

    pltpu.sync_copy(deg_out.at[pl.ds(r0, RPT)], dbuf)

    def _no_chunk(i, _):
        v = dbuf[pl.ds(i * 16, 16)] + 1.0
        no_ref[pl.ds(i * 16, 16)] = _rsqrt16(v)
        return _
    lax.fori_loop(0, RPT // 16, _no_chunk, None)
    pltpu.sync_copy(deg_in.at[pl.ds(r0, RPT)], dbuf)

    def _ni_chunk(i, _):
        v = dbuf[pl.ds(i * 16, 16)] + 1.0
        ni = _rsqrt16(v)
        ni_ref[pl.ds(i * 16, 16)] = ni
        cvec[pl.ds(i * 16, 16)] = ni * no_ref[pl.ds(i * 16, 16)]
        return _
    lax.fori_loop(0, RPT // 16, _ni_chunk, None)

    def _dense(scale_ref, from_agg, to_slab, to_out):
        xbv = gbuf.at[pl.ds(0, RC)]

        def _chunk(i, _):
            st = i * RC
            if from_agg:
                pltpu.sync_copy(agg.at[pl.ds(r0 + st, RC)], xbv)
            else:
                pltpu.sync_copy(x_flat.at[pl.ds(cbase + r0 + st, RC)], xbv)

            def _rowgrp(g, __):
                svec = scale_ref[pl.ds(st + g * 16, 16)]
                for r16 in range(16):
                    s = svec[r16]
                    r = g * 16 + r16
                    for c8 in range(H // 16):
                        gbuf[r, pl.ds(c8 * 16, 16)] = (
                            gbuf[r, pl.ds(c8 * 16, 16)] * s)
                return __
            lax.fori_loop(0, RC // 16, _rowgrp, None)
            if to_slab:
                pltpu.sync_copy(xbv, h_hbm.at[pl.ds(cbase + r0 + st, RC)])
                pltpu.sync_copy(xbv, agg.at[pl.ds(r0 + st, RC)])
            if to_out:
                pltpu.sync_copy(xbv, out_hbm.at[pl.ds(cbase + r0 + st, RC)])
            return _
        lax.fori_loop(0, NRC, _chunk, None)

    _dense(no_ref, from_agg=False, to_slab=True, to_out=False)
    plsc.subcore_barrier()

    for k in range(K):
        def _edge_chunk(j, _):
            pltpu.async_copy(
                h_hbm.at[sstage.at[pl.ds(j * CH, CH)]], gbuf, sem).wait()
            pltpu.sync_copy(gbuf, agg.at[dst2d.at[j]], add=True)
            return _
        lax.fori_loop(0, NCHUNK, _edge_chunk, None)
        plsc.subcore_barrier()
        if k < K - 1:
            _dense(cvec, from_agg=True, to_slab=True, to_out=False)
        else:
            _dense(ni_ref, from_agg=True, to_slab=False, to_out=True)
        plsc.subcore_barrier()


@jax.jit
def kernel(x, edge_index):
    f32 = jnp.float32
    mesh = plsc.VectorSubcoreMesh(core_axis_name="c", subcore_axis_name="s")
    run = pl.kernel(
        _sc_kprop,
        out_type=(
            jax.ShapeDtypeStruct((2 * NPAD, H), f32),
            jax.ShapeDtypeStruct((2 * NPAD, H), f32),
        ),
        mesh=mesh,
        scratch_types=[
            pltpu.VMEM((EPT_PAD,), jnp.int32),
            pltpu.VMEM((NCHUNK, CH), jnp.int32),
            pltpu.VMEM((CH,), jnp.int32),
            pltpu.VMEM((1, CH), jnp.int32),
            pltpu.VMEM((CH, H), f32),
            pltpu.VMEM((RPT,), f32),
            pltpu.VMEM((RPT,), f32),
            pltpu.VMEM((RPT,), f32),
            pltpu.VMEM((RPT,), f32),
            pltpu.VMEM((CH,), f32),
            pltpu.VMEM_SHARED((NT, H), f32),
            pltpu.VMEM_SHARED((NT,), f32),
            pltpu.VMEM_SHARED((NT,), f32),
            pltpu.SemaphoreType.DMA,
        ],
    )
    xt = x.reshape(N, 2, H).transpose(1, 0, 2)
    xp = jnp.pad(xt, ((0, 0), (0, NPAD - N), (0, 0))).reshape(2 * NPAD, H)
    out_flat, _ = run(xp, edge_index[0], edge_index[1])
    out = out_flat.reshape(2, NPAD, H)[:, :N, :]
    return out.transpose(1, 0, 2).reshape(N, D)


# speedup vs baseline: 7.6478x; 1.2963x over previous
import jax
import jax.numpy as jnp
from jax import lax
from jax.experimental import pallas as pl
from jax.experimental.pallas import tpu as pltpu
from jax.experimental.pallas import tpu_sc as plsc

N = 10000
E = 160000
D = 256
K = 2
H = D // 2
NS = 16
NW = 2 * NS
NPAD = 10240
EPT = E // NS
CH = 128
NCHUNK = 80
EPT_PAD = NCHUNK * CH
RPT = NPAD // NS
RC = 64
NRC = RPT // RC
EG = 512
NFG = 19
TAIL = EPT - NFG * EG


def _rsqrt16(v):
    i = lax.bitcast_convert_type(v, jnp.int32)
    i = 0x5F3759DF - (i >> 1)
    y = lax.bitcast_convert_type(i, jnp.float32)
    for _ in range(3):
        y = y * (1.5 - 0.5 * v * y * y)
    return y


def _sc_kprop(x_flat, src_hbm, dst_hbm, out_hbm, h_hbm, sidx_hbm, didx_hbm,
              tmps, tmpd, bs, bd, sidx_v, didx_v, gbufa, gbufb,
              no_ref, ni_ref, cvec, ones,
              agg, deg_out, deg_in, sema, semb, semi, semd):
    cid = lax.axis_index("c")
    sid = lax.axis_index("s")
    lane = lax.broadcasted_iota(jnp.int32, (16,), 0)
    trash = N + (lane & 7)
    r0 = sid * RPT
    g0 = sid * EPT
    cbase = cid * NPAD
    wbase = (cid * NS + sid) * NCHUNK

    def _init_z(i, _):
        no_ref[pl.ds(i * 16, 16)] = jnp.zeros((16,), jnp.float32)
        return _
    lax.fori_loop(0, RPT // 16, _init_z, None)
    for t in range(8):
        ones[pl.ds(t * 16, 16)] = jnp.ones((16,), jnp.float32)
    pltpu.sync_copy(no_ref, deg_out.at[pl.ds(cbase + r0, RPT)])
    pltpu.sync_copy(no_ref, deg_in.at[pl.ds(r0, RPT)])

    plsc.subcore_barrier()

    def _chunk16(jj, t, off):
        sv = tmps[pl.ds(off, 16)]
        dv = tmpd[pl.ds(off, 16)]
        m = sv != dv
        bd[jj, pl.ds(t * 16, 16)] = jnp.where(m, dv, trash)
        bs[jj, pl.ds(t * 16, 16)] = cbase + jnp.where(m, sv, trash)

    def _deg_fire():
        for jj in range(EG // CH):
            pltpu.sync_copy(ones, deg_out.at[bs.at[jj]], add=True)
            pltpu.sync_copy(ones, deg_in.at[bd.at[jj]], add=True)

    def _deg_drain(n):
        del n

    def _grp(g, _):
        pltpu.sync_copy(src_hbm.at[pl.ds(g0 + g * EG, EG)], tmps)
        pltpu.sync_copy(dst_hbm.at[pl.ds(g0 + g * EG, EG)], tmpd)

        @pl.when(g > 0)
        def _():
            _deg_drain(2 * (EG // CH))
        for jj in range(EG // CH):
            for t in range(8):
                _chunk16(jj, t, jj * CH + t * 16)
        jb = wbase + g * (EG // CH)
        pltpu.sync_copy(bs, sidx_hbm.at[pl.ds(jb, EG // CH)])
        pltpu.sync_copy(bd, didx_hbm.at[pl.ds(jb, EG // CH)])
        _deg_fire()
        return _
    lax.fori_loop(0, NFG, _grp, None)
    pltpu.sync_copy(src_hbm.at[pl.ds(g0 + NFG * EG, TAIL)],
                    tmps.at[pl.ds(0, TAIL)])
    pltpu.sync_copy(dst_hbm.at[pl.ds(g0 + NFG * EG, TAIL)],
                    tmpd.at[pl.ds(0, TAIL)])
    _deg_drain(2 * (EG // CH))
    for jj in range(2):
        for t in range(8):
            _chunk16(jj, t, jj * CH + t * 16)
    _chunk16(2, 0, 2 * CH)
    for t in range(EPT // 16, EPT_PAD // 16):
        jj, c = t // 8 - 76, (t % 8) * 16
        bs[jj, pl.ds(c, 16)] = cbase + N + ((t - EPT // 16) * 16 + lane)
        bd[jj, pl.ds(c, 16)] = trash
    jb = wbase + NFG * (EG // CH)
    pltpu.sync_copy(bs, sidx_hbm.at[pl.ds(jb, EG // CH)])
    pltpu.sync_copy(bd, didx_hbm.at[pl.ds(jb, EG // CH)])
    _deg_fire()
    _deg_drain(2 * (EG // CH))

    plsc.subcore_barrier()

    pltpu.sync_copy(deg_out.at[pl.ds(cbase + r0, RPT)], no_ref)
    pltpu.sync_copy(deg_in.at[pl.ds(r0, RPT)], ni_ref)

    def _norm_chunk(i, _):
        no = _rsqrt16(no_ref[pl.ds(i * 16, 16)] + 1.0)
        ni = _rsqrt16(ni_ref[pl.ds(i * 16, 16)] + 1.0)
        no_ref[pl.ds(i * 16, 16)] = no
        ni_ref[pl.ds(i * 16, 16)] = ni
        cvec[pl.ds(i * 16, 16)] = no * ni
        return _
    lax.fori_loop(0, RPT // 16, _norm_chunk, None)

    def _dense(scale_ref, from_agg, to_slab, to_out):
        xbv = gbufa.at[pl.ds(0, RC)]

        def _chunk(i, _):
            st = i * RC
            if from_agg:
                pltpu.sync_copy(agg.at[pl.ds(r0 + st, RC)], xbv)
            else:
                pltpu.sync_copy(x_flat.at[pl.ds(cbase + r0 + st, RC)], xbv)

            def _rowgrp(g, __):
                svec = scale_ref[pl.ds(st + g * 16, 16)]
                for r16 in range(16):
                    s = svec[r16]
                    r = g * 16 + r16
                    for c8 in range(H // 16):
                        gbufa[r, pl.ds(c8 * 16, 16)] = (
                            gbufa[r, pl.ds(c8 * 16, 16)] * s)
                return __
            lax.fori_loop(0, RC // 16, _rowgrp, None)
            if to_slab:
                pltpu.sync_copy(xbv, h_hbm.at[pl.ds(cbase + r0 + st, RC)])
                pltpu.sync_copy(xbv, agg.at[pl.ds(r0 + st, RC)])
            if to_out:
                pltpu.sync_copy(xbv, out_hbm.at[pl.ds(cbase + r0 + st, RC)])
            return _
        lax.fori_loop(0, NRC, _chunk, None)

    _dense(no_ref, from_agg=False, to_slab=True, to_out=False)
    plsc.subcore_barrier()

    def _idx_load(j, slot, sem):
        pltpu.async_copy(sidx_hbm.at[wbase + j], sidx_v.at[slot], sem)
        pltpu.async_copy(didx_hbm.at[wbase + j], didx_v.at[slot], sem)

    def _idx_drain(n, sem):
        for _i in range(n):
            pltpu.make_async_copy(
                sidx_hbm.at[0], sidx_v.at[0], sem).wait()

    for k in range(K):
        _idx_load(0, 0, semi)
        _idx_load(1, 1, semi)
        _idx_drain(4, semi)
        pltpu.async_copy(h_hbm.at[sidx_v.at[0]], gbufa, sema)

        def _pair(p, _):
            j0 = p * 2
            pltpu.make_async_copy(h_hbm.at[sidx_v.at[0]], gbufa, sema).wait()
            pltpu.async_copy(h_hbm.at[sidx_v.at[1]], gbufb, semb)
            pltpu.sync_copy(gbufa, agg.at[didx_v.at[0]], add=True)

            @pl.when(j0 + 2 < NCHUNK)
            def _():
                _idx_load(j0 + 2, 0, semi)
            pltpu.make_async_copy(h_hbm.at[sidx_v.at[0]], gbufb, semb).wait()

            @pl.when(j0 + 2 < NCHUNK)
            def _():
                _idx_drain(2, semi)
                pltpu.async_copy(h_hbm.at[sidx_v.at[0]], gbufa, sema)
            pltpu.sync_copy(gbufb, agg.at[didx_v.at[1]], add=True)

            @pl.when(j0 + 3 < NCHUNK)
            def _():
                _idx_load(j0 + 3, 1, semi)
                _idx_drain(2, semi)
            return _
        lax.fori_loop(0, NCHUNK // 2, _pair, None)
        plsc.subcore_barrier()
        if k < K - 1:
            _dense(cvec, from_agg=True, to_slab=True, to_out=False)
        else:
            _dense(ni_ref, from_agg=True, to_slab=False, to_out=True)
        plsc.subcore_barrier()


@jax.jit
def kernel(x, edge_index):
    f32 = jnp.float32
    mesh = plsc.VectorSubcoreMesh(core_axis_name="c", subcore_axis_name="s")
    run = pl.kernel(
        _sc_kprop,
        out_type=(
            jax.ShapeDtypeStruct((2 * NPAD, H), f32),
            jax.ShapeDtypeStruct((2 * NPAD, H), f32),
            jax.ShapeDtypeStruct((NW * NCHUNK, CH), jnp.int32),
            jax.ShapeDtypeStruct((NW * NCHUNK, CH), jnp.int32),
        ),
        mesh=mesh,
        scratch_types=[
            pltpu.VMEM((EG,), jnp.int32),
            pltpu.VMEM((EG,), jnp.int32),
            pltpu.VMEM((EG // CH, CH), jnp.int32),
            pltpu.VMEM((EG // CH, CH), jnp.int32),
            pltpu.VMEM((2, CH), jnp.int32),
            pltpu.VMEM((2, CH), jnp.int32),
            pltpu.VMEM((CH, H), f32),
            pltpu.VMEM((CH, H), f32),
            pltpu.VMEM((RPT,), f32),
            pltpu.VMEM((RPT,), f32),
            pltpu.VMEM((RPT,), f32),
            pltpu.VMEM((CH,), f32),
            pltpu.VMEM_SHARED((NPAD, H), f32),
            pltpu.VMEM_SHARED((2 * NPAD,), f32),
            pltpu.VMEM_SHARED((NPAD,), f32),
            pltpu.SemaphoreType.DMA,
            pltpu.SemaphoreType.DMA,
            pltpu.SemaphoreType.DMA,
            pltpu.SemaphoreType.DMA,
        ],
    )
    xt = x.reshape(N, 2, H).transpose(1, 0, 2)
    xp = jnp.pad(xt, ((0, 0), (0, NPAD - N), (0, 0))).reshape(2 * NPAD, H)
    out_flat, _, _, _ = run(xp, edge_index[0], edge_index[1])
    out = out_flat.reshape(2, NPAD, H)[:, :N, :]
    return out.transpose(1, 0, 2).reshape(N, D)


# speedup vs baseline: 8.3030x; 1.0857x over previous
import jax
import jax.numpy as jnp
from jax import lax
from jax.experimental import pallas as pl
from jax.experimental.pallas import tpu as pltpu
from jax.experimental.pallas import tpu_sc as plsc

N = 10000
E = 160000
D = 256
K = 2
H = D // 2
NS = 16
NW = 2 * NS
NPAD = 10240
EPT = E // NS
CH = 128
NCHUNK = 80
EPT_PAD = NCHUNK * CH
RPT = NPAD // NS
RC = 64
NRC = RPT // RC
EG = 512
NFG = 19
TAIL = EPT - NFG * EG


def _rsqrt16(v):
    i = lax.bitcast_convert_type(v, jnp.int32)
    i = 0x5F3759DF - (i >> 1)
    y = lax.bitcast_convert_type(i, jnp.float32)
    for _ in range(3):
        y = y * (1.5 - 0.5 * v * y * y)
    return y


def _sc_kprop(x_flat, src_hbm, dst_hbm, out_hbm, h_hbm, sidx_hbm, didx_hbm,
              tmps, tmpd, bs, bd, sidx_v, didx_v, gbufa, gbufb,
              no_ref, ni_ref, cvec, ones,
              agg, deg_out, deg_in, sema, semb, semi, semd, semw):
    cid = lax.axis_index("c")
    sid = lax.axis_index("s")
    lane = lax.broadcasted_iota(jnp.int32, (16,), 0)
    trash = N + (lane & 7)
    r0 = sid * RPT
    g0 = sid * EPT
    cbase = cid * NPAD
    wbase = (cid * NS + sid) * NCHUNK

    def _init_z(i, _):
        no_ref[pl.ds(i * 16, 16)] = jnp.zeros((16,), jnp.float32)
        return _
    lax.fori_loop(0, RPT // 16, _init_z, None)
    for t in range(8):
        ones[pl.ds(t * 16, 16)] = jnp.ones((16,), jnp.float32)
    pltpu.sync_copy(no_ref, deg_out.at[pl.ds(cbase + r0, RPT)])
    pltpu.sync_copy(no_ref, deg_in.at[pl.ds(r0, RPT)])

    plsc.subcore_barrier()

    NR = EG // CH

    def _chunk16(sl, jj, t, off):
        sv = tmps[pl.ds(sl * EG + off, 16)]
        dv = tmpd[pl.ds(sl * EG + off, 16)]
        m = sv != dv
        bd[sl * NR + jj, pl.ds(t * 16, 16)] = jnp.where(m, dv, trash)
        bs[sl * NR + jj, pl.ds(t * 16, 16)] = (
            cbase + jnp.where(m, sv, trash))

    def _deg_pair(sl, jj):
        pltpu.async_copy(ones, deg_out.at[bs.at[sl * NR + jj]], semd,
                         add=True)
        pltpu.async_copy(ones, deg_in.at[bd.at[sl * NR + jj]], semd,
                         add=True)
        for _i in range(2):
            pltpu.make_async_copy(
                ones, deg_in.at[pl.ds(0, CH)], semd).wait()

    def _ld_grp(g, sl, sem):
        pltpu.async_copy(src_hbm.at[pl.ds(g0 + g * EG, EG)],
                         tmps.at[pl.ds(sl * EG, EG)], sem)
        pltpu.async_copy(dst_hbm.at[pl.ds(g0 + g * EG, EG)],
                         tmpd.at[pl.ds(sl * EG, EG)], sem)

    def _drain_ld(n):
        for _i in range(n):
            pltpu.make_async_copy(
                src_hbm.at[pl.ds(0, EG)], tmps.at[pl.ds(0, EG)], semi).wait()

    def _drain_wr(n):
        for _i in range(n):
            pltpu.make_async_copy(
                bs.at[pl.ds(0, NR)], sidx_hbm.at[pl.ds(wbase, NR)],
                semw).wait()

    _ld_grp(0, 0, semi)
    _drain_ld(2)

    def _grp(g, _):
        sl = g & 1

        @pl.when(g + 1 < NFG)
        def _():
            _ld_grp(g + 1, 1 - sl, semi)

        @pl.when(g > 1)
        def _():
            _drain_wr(2)
        for jj in range(NR):
            for t in range(8):
                _chunk16(sl, jj, t, jj * CH + t * 16)
        jb = wbase + g * NR
        pltpu.async_copy(bs.at[pl.ds(sl * NR, NR)],
                         sidx_hbm.at[pl.ds(jb, NR)], semw)
        pltpu.async_copy(bd.at[pl.ds(sl * NR, NR)],
                         didx_hbm.at[pl.ds(jb, NR)], semw)
        for jj in range(NR):
            _deg_pair(sl, jj)

        @pl.when(g + 1 < NFG)
        def _():
            _drain_ld(2)
        return _
    lax.fori_loop(0, NFG, _grp, None)
    _drain_wr(4)
    pltpu.sync_copy(src_hbm.at[pl.ds(g0 + NFG * EG, TAIL)],
                    tmps.at[pl.ds(0, TAIL)])
    pltpu.sync_copy(dst_hbm.at[pl.ds(g0 + NFG * EG, TAIL)],
                    tmpd.at[pl.ds(0, TAIL)])
    for jj in range(2):
        for t in range(8):
            _chunk16(0, jj, t, jj * CH + t * 16)
    _chunk16(0, 2, 0, 2 * CH)
    for t in range(EPT // 16, EPT_PAD // 16):
        jj, c = t // 8 - 76, (t % 8) * 16
        bs[jj, pl.ds(c, 16)] = cbase + N + ((t - EPT // 16) * 16 + lane)
        bd[jj, pl.ds(c, 16)] = trash
    jb = wbase + NFG * NR
    pltpu.sync_copy(bs.at[pl.ds(0, NR)], sidx_hbm.at[pl.ds(jb, NR)])
    pltpu.sync_copy(bd.at[pl.ds(0, NR)], didx_hbm.at[pl.ds(jb, NR)])
    for jj in range(NR):
        _deg_pair(0, jj)

    plsc.subcore_barrier()

    pltpu.sync_copy(deg_out.at[pl.ds(cbase + r0, RPT)], no_ref)
    pltpu.sync_copy(deg_in.at[pl.ds(r0, RPT)], ni_ref)

    def _norm_chunk(i, _):
        no = _rsqrt16(no_ref[pl.ds(i * 16, 16)] + 1.0)
        ni = _rsqrt16(ni_ref[pl.ds(i * 16, 16)] + 1.0)
        no_ref[pl.ds(i * 16, 16)] = no
        ni_ref[pl.ds(i * 16, 16)] = ni
        cvec[pl.ds(i * 16, 16)] = no * ni
        return _
    lax.fori_loop(0, RPT // 16, _norm_chunk, None)

    def _dense(scale_ref, from_agg, to_slab, to_out):
        xbv = gbufa.at[pl.ds(0, RC)]

        def _wdrain():
            pltpu.make_async_copy(
                xbv, h_hbm.at[pl.ds(cbase + r0, RC)], semw).wait()
            if to_slab:
                pltpu.make_async_copy(
                    xbv, agg.at[pl.ds(r0, RC)], semd).wait()

        def _chunk(i, _):
            st = i * RC

            @pl.when(i > 0)
            def _():
                _wdrain()
            if from_agg:
                pltpu.sync_copy(agg.at[pl.ds(r0 + st, RC)], xbv)
            else:
                pltpu.sync_copy(x_flat.at[pl.ds(cbase + r0 + st, RC)], xbv)

            def _rowgrp(g, __):
                svec = scale_ref[pl.ds(st + g * 16, 16)]
                for r16 in range(16):
                    s = svec[r16]
                    r = g * 16 + r16
                    for c8 in range(H // 16):
                        gbufa[r, pl.ds(c8 * 16, 16)] = (
                            gbufa[r, pl.ds(c8 * 16, 16)] * s)
                return __
            lax.fori_loop(0, RC // 16, _rowgrp, None)
            if to_slab:
                pltpu.async_copy(
                    xbv, h_hbm.at[pl.ds(cbase + r0 + st, RC)], semw)
                pltpu.async_copy(xbv, agg.at[pl.ds(r0 + st, RC)], semd)
            if to_out:
                pltpu.async_copy(
                    xbv, out_hbm.at[pl.ds(cbase + r0 + st, RC)], semw)
            return _
        lax.fori_loop(0, NRC, _chunk, None)
        _wdrain()

    _dense(no_ref, from_agg=False, to_slab=True, to_out=False)
    plsc.subcore_barrier()

    def _idx_load(j, slot, sem):
        pltpu.async_copy(sidx_hbm.at[wbase + j], sidx_v.at[slot], sem)
        pltpu.async_copy(didx_hbm.at[wbase + j], didx_v.at[slot], sem)

    def _idx_drain(n, sem):
        for _i in range(n):
            pltpu.make_async_copy(
                sidx_hbm.at[0], sidx_v.at[0], sem).wait()

    for k in range(K):
        _idx_load(0, 0, semi)
        _idx_load(1, 1, semi)
        _idx_drain(4, semi)
        pltpu.async_copy(h_hbm.at[sidx_v.at[0]], gbufa, sema)

        def _pair(p, _):
            j0 = p * 2
            pltpu.make_async_copy(h_hbm.at[sidx_v.at[0]], gbufa, sema).wait()
            pltpu.async_copy(h_hbm.at[sidx_v.at[1]], gbufb, semb)
            pltpu.sync_copy(gbufa, agg.at[didx_v.at[0]], add=True)

            @pl.when(j0 + 2 < NCHUNK)
            def _():
                _idx_load(j0 + 2, 0, semi)
            pltpu.make_async_copy(h_hbm.at[sidx_v.at[0]], gbufb, semb).wait()

            @pl.when(j0 + 2 < NCHUNK)
            def _():
                _idx_drain(2, semi)
                pltpu.async_copy(h_hbm.at[sidx_v.at[0]], gbufa, sema)
            pltpu.sync_copy(gbufb, agg.at[didx_v.at[1]], add=True)

            @pl.when(j0 + 3 < NCHUNK)
            def _():
                _idx_load(j0 + 3, 1, semi)
                _idx_drain(2, semi)
            return _
        lax.fori_loop(0, NCHUNK // 2, _pair, None)
        plsc.subcore_barrier()
        if k < K - 1:
            _dense(cvec, from_agg=True, to_slab=True, to_out=False)
        else:
            _dense(ni_ref, from_agg=True, to_slab=False, to_out=True)
        plsc.subcore_barrier()


@jax.jit
def kernel(x, edge_index):
    f32 = jnp.float32
    mesh = plsc.VectorSubcoreMesh(core_axis_name="c", subcore_axis_name="s")
    run = pl.kernel(
        _sc_kprop,
        out_type=(
            jax.ShapeDtypeStruct((2 * NPAD, H), f32),
            jax.ShapeDtypeStruct((2 * NPAD, H), f32),
            jax.ShapeDtypeStruct((NW * NCHUNK, CH), jnp.int32),
            jax.ShapeDtypeStruct((NW * NCHUNK, CH), jnp.int32),
        ),
        mesh=mesh,
        scratch_types=[
            pltpu.VMEM((2 * EG,), jnp.int32),
            pltpu.VMEM((2 * EG,), jnp.int32),
            pltpu.VMEM((2 * (EG // CH), CH), jnp.int32),
            pltpu.VMEM((2 * (EG // CH), CH), jnp.int32),
            pltpu.VMEM((2, CH), jnp.int32),
            pltpu.VMEM((2, CH), jnp.int32),
            pltpu.VMEM((CH, H), f32),
            pltpu.VMEM((CH, H), f32),
            pltpu.VMEM((RPT,), f32),
            pltpu.VMEM((RPT,), f32),
            pltpu.VMEM((RPT,), f32),
            pltpu.VMEM((CH,), f32),
            pltpu.VMEM_SHARED((NPAD, H), f32),
            pltpu.VMEM_SHARED((2 * NPAD,), f32),
            pltpu.VMEM_SHARED((NPAD,), f32),
            pltpu.SemaphoreType.DMA,
            pltpu.SemaphoreType.DMA,
            pltpu.SemaphoreType.DMA,
            pltpu.SemaphoreType.DMA,
            pltpu.SemaphoreType.DMA,
        ],
    )
    xt = x.reshape(N, 2, H).transpose(1, 0, 2)
    xp = jnp.pad(xt, ((0, 0), (0, NPAD - N), (0, 0))).reshape(2 * NPAD, H)
    out_flat, _, _, _ = run(xp, edge_index[0], edge_index[1])
    out = out_flat.reshape(2, NPAD, H)[:, :N, :]
    return out.transpose(1, 0, 2).reshape(N, D)
